# Initial kernel scaffold; baseline (speedup 1.0000x reference)
import jax
import jax.numpy as jnp
from jax.experimental import pallas as pl


def kernel(X, adj, table):
    raise NotImplementedError("write your pallas kernel here")


# speedup vs baseline: 6.6583x; 6.6583x over previous
import jax
import jax.numpy as jnp
from jax import lax
from jax.experimental import pallas as pl
from jax.experimental.pallas import tpu as pltpu
from jax.experimental.pallas import tpu_sc as plsc

N_NODES = 100000
K = 32
D = 128
B = 16384

NC = 2
NS = 16
NW = NC * NS
BPW = B // NW
C = 2
NCHUNK = BPW // C
LANES = 16
NV = D // LANES
INV_K = 1.0 / K
ISLC = 128


def _gcn_body(x_hbm, adj_hbm, table_hbm, out_hbm,
              x_v, edge_v, self_v, nb0, nb1, out_v,
              sem_e, sem_s, sem_n0, sem_n1):
    wid = lax.axis_index("s") * NC + lax.axis_index("c")
    base = wid * BPW

    pltpu.sync_copy(x_hbm.at[pl.ds(base, BPW)], x_v)

    for j in range(BPW // ISLC):
        sl = pl.ds(j * ISLC, ISLC)
        pltpu.async_copy(adj_hbm.at[x_v.at[sl]], edge_v.at[sl], sem_e)
    for j in range(BPW // ISLC):
        sl = pl.ds(j * ISLC, ISLC)
        pltpu.async_copy(table_hbm.at[x_v.at[sl]], self_v.at[sl], sem_s)
    for j in range(BPW // ISLC):
        sl = pl.ds(j * ISLC, ISLC)
        pltpu.make_async_copy(adj_hbm.at[x_v.at[sl]], edge_v.at[sl], sem_e).wait()

    def fire(g, nb, sem):
        for q in range(C):
            pltpu.async_copy(table_hbm.at[edge_v.at[g * C + q]], nb.at[q], sem)

    def drain(g, nb, sem):
        for q in range(C):
            pltpu.make_async_copy(
                table_hbm.at[edge_v.at[g * C + q]], nb.at[q], sem).wait()

    def compute(g, nb):
        for q in range(C):
            qi = g * C + q
            accs = [nb[q, 0, pl.ds(d * LANES, LANES)] for d in range(NV)]
            for k in range(1, K):
                for d in range(NV):
                    accs[d] = accs[d] + nb[q, k, pl.ds(d * LANES, LANES)]
            for d in range(NV):
                dsl = pl.ds(d * LANES, LANES)
                out_v[q, dsl] = accs[d] * INV_K + self_v[qi, dsl]
        pltpu.sync_copy(out_v, out_hbm.at[pl.ds(base + g * C, C)])

    fire(0, nb0, sem_n0)
    for j in range(BPW // ISLC):
        sl = pl.ds(j * ISLC, ISLC)
        pltpu.make_async_copy(table_hbm.at[x_v.at[sl]], self_v.at[sl], sem_s).wait()

    def step(i, carry):
        g0 = 2 * i
        fire(g0 + 1, nb1, sem_n1)
        drain(g0, nb0, sem_n0)
        compute(g0, nb0)

        @pl.when(g0 + 2 < NCHUNK)
        def _():
            fire(g0 + 2, nb0, sem_n0)

        drain(g0 + 1, nb1, sem_n1)
        compute(g0 + 1, nb1)
        return carry

    lax.fori_loop(0, NCHUNK // 2, step, 0)


def kernel(X, adj, table):
    x = jnp.reshape(X, (B,)).astype(jnp.int32)
    adj32 = adj.astype(jnp.int32)
    f = pl.kernel(
        _gcn_body,
        out_type=jax.ShapeDtypeStruct((B, D), jnp.float32),
        mesh=plsc.VectorSubcoreMesh(core_axis_name="c", subcore_axis_name="s"),
        compiler_params=pltpu.CompilerParams(use_tc_tiling_on_sc=False),
        scratch_types=[
            pltpu.VMEM((BPW,), jnp.int32),
            pltpu.VMEM((BPW, K), jnp.int32),
            pltpu.VMEM((BPW, D), jnp.float32),
            pltpu.VMEM((C, K, D), jnp.float32),
            pltpu.VMEM((C, K, D), jnp.float32),
            pltpu.VMEM((C, D), jnp.float32),
            pltpu.SemaphoreType.DMA,
            pltpu.SemaphoreType.DMA,
            pltpu.SemaphoreType.DMA,
            pltpu.SemaphoreType.DMA,
        ],
    )
    out = f(x, adj32, table)
    return jnp.reshape(out, (B, 1, D))
